# baseline (speedup 1.0000x reference)
import functools

import jax
import jax.numpy as jnp
from jax import lax
from jax.experimental import pallas as pl
from jax.experimental.pallas import tpu as pltpu
from jax.experimental.pallas import tpu_sc as plsc

RES1 = 513
CH = 32
NPTS = 262144
HW = RES1 * RES1
LAST = RES1 - 1

L = 16
NC, NS = 2, 16
NW = NC * NS
PPW = NPTS // NW
B = 64
NCHUNK = PPW // B

WLEN = 512
NWIN = (HW - 1) // WLEN
HWP = ((HW + 7) // 8) * 8

PLANE_AXES = ((0, 1), (1, 2), (0, 2))


def _tri_plane_body(xyz, xy, yz, xz, out, tpriv,
                    tin0, tin1, tbuf,
                    xyzv0, xyzv1, w0, w1, idx0, idx1, rows0, rows1,
                    outv0, outv1,
                    semt0, semt1,
                    semg0, semg1, semx0, semx1, semo0, semo1):
    tin = (tin0, tin1)
    xyzv = (xyzv0, xyzv1)
    wv = (w0, w1)
    idxv = (idx0, idx1)
    rowsv = (rows0, rows1)
    outv = (outv0, outv1)
    semt = (semt0, semt1)
    semg = (semg0, semg1)
    semx = (semx0, semx1)
    semo = (semo0, semo1)

    cid = lax.axis_index("c")
    sid = lax.axis_index("s")
    wid = sid * NC + cid
    wbase = wid * PPW

    planes = (xy, yz, xz)
    tbase = cid * (3 * HWP)

    def transpose_win(n_pix, q):
        @plsc.parallel_loop(0, n_pix // L, 1, unroll=2)
        def _grp(g):
            pix = g * L + lax.iota(jnp.int32, L)
            for c in range(CH):
                v = tin[q][c, pl.ds(g * L, L)]
                plsc.store_scatter(
                    tbuf, [pix, jnp.full((L,), c, jnp.int32)], v)

    for p in range(3):
        plane = planes[p]

        def stage(j, q, plane=plane):
            pltpu.async_copy(
                plane.at[:, pl.ds(j * WLEN, WLEN)], tin[q].at[:, :WLEN],
                semt[q])

        def wait_stage(j, q, plane=plane):
            pltpu.make_async_copy(
                plane.at[:, pl.ds(j * WLEN, WLEN)], tin[q].at[:, :WLEN],
                semt[q]).wait()

        stage(sid, 0)

        def t_pair(h, c2, p=p, stage=stage, wait_stage=wait_stage):
            for par in (0, 1):
                k = 2 * h + par
                j = sid + NS * k

                @pl.when(j < NWIN)
                def _():
                    jn = j + NS

                    @pl.when(jn < NWIN)
                    def _():
                        stage(jn, 1 - par)

                    wait_stage(j, par)
                    transpose_win(WLEN, par)
                    pltpu.sync_copy(
                        tbuf,
                        tpriv.at[pl.ds(tbase + p * HWP + j * WLEN, WLEN)])
            return c2

        nk = (NWIN + NS - 1) // NS
        lax.fori_loop(0, (nk + 1) // 2, t_pair, 0, unroll=False)

        @pl.when(sid == 0)
        def _(p=p, plane=plane):
            pltpu.sync_copy(
                plane.at[:, pl.ds(NWIN * WLEN, 1)], tin[0].at[:, :1])
            pix = lax.iota(jnp.int32, L)
            m = pix < 1
            for c in range(CH):
                v = tin[0][c, pl.ds(0, L)]
                plsc.store_scatter(
                    tbuf, [pix, jnp.full((L,), c, jnp.int32)], v, mask=m)
            pltpu.sync_copy(
                tbuf.at[pl.ds(0, 8)],
                tpriv.at[pl.ds(tbase + p * HWP + NWIN * WLEN, 8)])

    plsc.subcore_barrier()

    def phase_a(q):
        @plsc.parallel_loop(0, B // L, 1, unroll=2)
        def idx_body(g):
            sl = pl.ds(g * L, L)
            rvec = g * L + lax.iota(jnp.int32, L)
            i0s, i1s = [], []
            for d in range(3):
                v = plsc.load_gather(
                    xyzv[q], [rvec, jnp.full((L,), d, jnp.int32)])
                f = jnp.clip((v + 1.0) * (0.5 * LAST), 0.0, float(LAST))
                i0 = f.astype(jnp.int32)
                wv[q][d, sl] = f - i0.astype(jnp.float32)
                i0s.append(i0)
                i1s.append(jnp.minimum(i0 + 1, LAST))
            pbase = cid * (3 * HWP)
            for p, (a, b) in enumerate(PLANE_AXES):
                rr0 = i0s[b] * RES1 + pbase + (p * HWP)
                rr1 = i1s[b] * RES1 + pbase + (p * HWP)
                idxv[q][4 * p + 0, sl] = rr0 + i0s[a]
                idxv[q][4 * p + 1, sl] = rr0 + i1s[a]
                idxv[q][4 * p + 2, sl] = rr1 + i0s[a]
                idxv[q][4 * p + 3, sl] = rr1 + i1s[a]

    def gather_copies(q):
        return [
            pltpu.make_async_copy(
                tpriv.at[idxv[q].at[j]], rowsv[q].at[j], semg[q])
            for j in range(12)
        ]

    def blend(q):
        @plsc.parallel_loop(0, B, 1, unroll=4)
        def _pt(t):
            tvec = jnp.full((L,), t, jnp.int32)
            w = [
                plsc.load_gather(
                    wv[q], [jnp.full((L,), d, jnp.int32), tvec])
                for d in range(3)
            ]
            for p, (a, b) in enumerate(PLANE_AXES):
                wa = w[a]
                wb = w[b]
                for h in range(CH // L):
                    cs = pl.ds(h * L, L)
                    v00 = rowsv[q][4 * p + 0, t, cs]
                    v01 = rowsv[q][4 * p + 1, t, cs]
                    v10 = rowsv[q][4 * p + 2, t, cs]
                    v11 = rowsv[q][4 * p + 3, t, cs]
                    top = v00 + wa * (v01 - v00)
                    bot = v10 + wa * (v11 - v10)
                    outv[q][t, pl.ds(p * CH + h * L, L)] = (
                        top + wb * (bot - top))

    pltpu.sync_copy(xyz.at[pl.ds(wbase, B)], xyzv[0])
    phase_a(0)
    for cp in gather_copies(0):
        cp.start()
    pltpu.async_copy(xyz.at[pl.ds(wbase + B, B)], xyzv[1], semx[1])

    def one_chunk(k, par):
        opar = 1 - par

        @pl.when(k + 1 < NCHUNK)
        def _():
            pltpu.make_async_copy(
                xyz.at[pl.ds(wbase + (k + 1) * B, B)], xyzv[opar],
                semx[opar]).wait()
            phase_a(opar)
            for cp in gather_copies(opar):
                cp.start()

        @pl.when(k + 2 < NCHUNK)
        def _():
            pltpu.async_copy(
                xyz.at[pl.ds(wbase + (k + 2) * B, B)], xyzv[par], semx[par])

        for cp in gather_copies(par):
            cp.wait()

        blend(par)

        @pl.when(k >= 2)
        def _():
            pltpu.make_async_copy(
                outv[par], out.at[pl.ds(wbase + (k - 2) * B, B)],
                semo[par]).wait()

        pltpu.async_copy(
            outv[par], out.at[pl.ds(wbase + k * B, B)], semo[par])

    def pair_body(k2, carry):
        one_chunk(2 * k2, 0)
        one_chunk(2 * k2 + 1, 1)
        return carry

    lax.fori_loop(0, NCHUNK // 2, pair_body, 0, unroll=False)

    pltpu.make_async_copy(
        outv[0], out.at[pl.ds(wbase + (NCHUNK - 2) * B, B)], semo[0]).wait()
    pltpu.make_async_copy(
        outv[1], out.at[pl.ds(wbase + (NCHUNK - 1) * B, B)], semo[1]).wait()


@jax.jit
def _tri_plane_sc(xyz, xy, yz, xz):
    mesh = plsc.VectorSubcoreMesh(core_axis_name="c", subcore_axis_name="s")
    out, _ = pl.kernel(
        _tri_plane_body,
        mesh=mesh,
        compiler_params=pltpu.CompilerParams(
            needs_layout_passes=False, use_tc_tiling_on_sc=False,
            disable_bounds_checks=True),
        out_type=(
            jax.ShapeDtypeStruct((NPTS, 3 * CH), jnp.float32),
            jax.ShapeDtypeStruct((NC * 3 * HWP, CH), jnp.float32),
        ),
        scratch_types=[
            pltpu.VMEM((CH, WLEN), jnp.float32),
            pltpu.VMEM((CH, WLEN), jnp.float32),
            pltpu.VMEM((WLEN, CH), jnp.float32),
            pltpu.VMEM((B, 3), jnp.float32),
            pltpu.VMEM((B, 3), jnp.float32),
            pltpu.VMEM((3, B), jnp.float32),
            pltpu.VMEM((3, B), jnp.float32),
            pltpu.VMEM((12, B), jnp.int32),
            pltpu.VMEM((12, B), jnp.int32),
            pltpu.VMEM((12, B, CH), jnp.float32),
            pltpu.VMEM((12, B, CH), jnp.float32),
            pltpu.VMEM((B, 3 * CH), jnp.float32),
            pltpu.VMEM((B, 3 * CH), jnp.float32),
            pltpu.SemaphoreType.DMA,
            pltpu.SemaphoreType.DMA,
            pltpu.SemaphoreType.DMA,
            pltpu.SemaphoreType.DMA,
            pltpu.SemaphoreType.DMA,
            pltpu.SemaphoreType.DMA,
            pltpu.SemaphoreType.DMA,
            pltpu.SemaphoreType.DMA,
        ],
    )(xyz, xy, yz, xz)
    return out


def kernel(xyz, xy, yz, xz):
    return _tri_plane_sc(
        xyz, xy.reshape(CH, HW), yz.reshape(CH, HW), xz.reshape(CH, HW))


# speedup vs baseline: 2.2257x; 2.2257x over previous
import functools

import jax
import jax.numpy as jnp
from jax import lax
from jax.experimental import pallas as pl
from jax.experimental.pallas import tpu as pltpu
from jax.experimental.pallas import tpu_sc as plsc

RES1 = 513
CH = 32
NPTS = 262144
HW = RES1 * RES1
LAST = RES1 - 1

L = 16
NC, NS = 2, 16
NW = NC * NS
PPW = NPTS // NW
B = 64
NCHUNK = PPW // B

PLANE_AXES = ((0, 1), (1, 2), (0, 2))


def _tri_plane_body(xyz, table, out,
                    xyzv0, xyzv1, w0, w1, idx0, idx1, rows0, rows1,
                    outv0, outv1,
                    semg0, semg1, semx0, semx1, semo0, semo1):
    xyzv = (xyzv0, xyzv1)
    wv = (w0, w1)
    idxv = (idx0, idx1)
    rowsv = (rows0, rows1)
    outv = (outv0, outv1)
    semg = (semg0, semg1)
    semx = (semx0, semx1)
    semo = (semo0, semo1)

    wid = lax.axis_index("s") * NC + lax.axis_index("c")
    wbase = wid * PPW

    def phase_a(q):
        @plsc.parallel_loop(0, B // L, 1, unroll=2)
        def idx_body(g):
            sl = pl.ds(g * L, L)
            rvec = g * L + lax.iota(jnp.int32, L)
            i0s, i1s = [], []
            for d in range(3):
                v = plsc.load_gather(
                    xyzv[q], [rvec, jnp.full((L,), d, jnp.int32)])
                f = jnp.clip((v + 1.0) * (0.5 * LAST), 0.0, float(LAST))
                i0 = f.astype(jnp.int32)
                wv[q][d, sl] = f - i0.astype(jnp.float32)
                i0s.append(i0)
                i1s.append(jnp.minimum(i0 + 1, LAST))
            for p, (a, b) in enumerate(PLANE_AXES):
                r0 = i0s[b] * RES1 + (p * HW)
                r1 = i1s[b] * RES1 + (p * HW)
                idxv[q][4 * p + 0, sl] = r0 + i0s[a]
                idxv[q][4 * p + 1, sl] = r0 + i1s[a]
                idxv[q][4 * p + 2, sl] = r1 + i0s[a]
                idxv[q][4 * p + 3, sl] = r1 + i1s[a]

    def gather_copies(q, table_ref):
        return [
            pltpu.make_async_copy(
                table_ref.at[idxv[q].at[j]], rowsv[q].at[j], semg[q])
            for j in range(12)
        ]

    def blend(q):
        @plsc.parallel_loop(0, B, 1, unroll=4)
        def _pt(t):
            tvec = jnp.full((L,), t, jnp.int32)
            w = [
                plsc.load_gather(
                    wv[q], [jnp.full((L,), d, jnp.int32), tvec])
                for d in range(3)
            ]
            for p, (a, b) in enumerate(PLANE_AXES):
                wa = w[a]
                wb = w[b]
                for h in range(CH // L):
                    cs = pl.ds(h * L, L)
                    v00 = rowsv[q][4 * p + 0, t, cs]
                    v01 = rowsv[q][4 * p + 1, t, cs]
                    v10 = rowsv[q][4 * p + 2, t, cs]
                    v11 = rowsv[q][4 * p + 3, t, cs]
                    top = v00 + wa * (v01 - v00)
                    bot = v10 + wa * (v11 - v10)
                    outv[q][t, pl.ds(p * CH + h * L, L)] = (
                        top + wb * (bot - top))

    pltpu.sync_copy(xyz.at[pl.ds(wbase, B)], xyzv[0])
    phase_a(0)
    for cp in gather_copies(0, table):
        cp.start()
    pltpu.async_copy(xyz.at[pl.ds(wbase + B, B)], xyzv[1], semx[1])

    def one_chunk(k, par):
        opar = 1 - par

        @pl.when(k + 1 < NCHUNK)
        def _():
            pltpu.make_async_copy(
                xyz.at[pl.ds(wbase + (k + 1) * B, B)], xyzv[opar],
                semx[opar]).wait()
            phase_a(opar)
            for cp in gather_copies(opar, table):
                cp.start()

        @pl.when(k + 2 < NCHUNK)
        def _():
            pltpu.async_copy(
                xyz.at[pl.ds(wbase + (k + 2) * B, B)], xyzv[par], semx[par])

        for cp in gather_copies(par, table):
            cp.wait()

        blend(par)

        @pl.when(k >= 2)
        def _():
            pltpu.make_async_copy(
                outv[par], out.at[pl.ds(wbase + (k - 2) * B, B)],
                semo[par]).wait()

        pltpu.async_copy(
            outv[par], out.at[pl.ds(wbase + k * B, B)], semo[par])

    def pair_body(k2, carry):
        one_chunk(2 * k2, 0)
        one_chunk(2 * k2 + 1, 1)
        return carry

    lax.fori_loop(0, NCHUNK // 2, pair_body, 0, unroll=False)

    pltpu.make_async_copy(
        outv[0], out.at[pl.ds(wbase + (NCHUNK - 2) * B, B)], semo[0]).wait()
    pltpu.make_async_copy(
        outv[1], out.at[pl.ds(wbase + (NCHUNK - 1) * B, B)], semo[1]).wait()


@jax.jit
def _tri_plane_sc(xyz, table):
    mesh = plsc.VectorSubcoreMesh(core_axis_name="c", subcore_axis_name="s")
    return pl.kernel(
        _tri_plane_body,
        mesh=mesh,
        compiler_params=pltpu.CompilerParams(
            needs_layout_passes=False, use_tc_tiling_on_sc=False,
            disable_bounds_checks=True),
        out_type=jax.ShapeDtypeStruct((NPTS, 3 * CH), jnp.float32),
        scratch_types=[
            pltpu.VMEM((B, 3), jnp.float32),
            pltpu.VMEM((B, 3), jnp.float32),
            pltpu.VMEM((3, B), jnp.float32),
            pltpu.VMEM((3, B), jnp.float32),
            pltpu.VMEM((12, B), jnp.int32),
            pltpu.VMEM((12, B), jnp.int32),
            pltpu.VMEM((12, B, CH), jnp.float32),
            pltpu.VMEM((12, B, CH), jnp.float32),
            pltpu.VMEM((B, 3 * CH), jnp.float32),
            pltpu.VMEM((B, 3 * CH), jnp.float32),
            pltpu.SemaphoreType.DMA,
            pltpu.SemaphoreType.DMA,
            pltpu.SemaphoreType.DMA,
            pltpu.SemaphoreType.DMA,
            pltpu.SemaphoreType.DMA,
            pltpu.SemaphoreType.DMA,
        ],
    )(xyz, table)


def kernel(xyz, xy, yz, xz):
    table = (
        jnp.stack([xy, yz, xz])
        .transpose(0, 2, 3, 1)
        .reshape(3 * HW, CH)
    )
    return _tri_plane_sc(xyz, table)
